# baseline (speedup 1.0000x reference)
import functools
import jax
import jax.numpy as jnp
from jax import lax
from jax.experimental import pallas as pl
from jax.experimental.pallas import tpu as pltpu
from jax.experimental.pallas import tpu_sc as plsc

_TB = 1024
_CH = 128


def _tables_body(emb_ref, w1t_ref, b1_ref, w2t_ref, b2_ref, t1_ref, t2_ref):
    dn = (((1,), (0,)), ((), ()))
    t1 = jax.lax.dot_general(emb_ref[...], w1t_ref[...], dn,
                             preferred_element_type=jnp.float32) + b1_ref[0][None, :]
    t1_ref[...] = t1
    t2_ref[...] = jax.lax.dot_general(t1, w2t_ref[...], dn,
                                      preferred_element_type=jnp.float32) + b2_ref[0][None, :]


def _hid_body(ids_ref, t1_ref, t2_ref, h1_ref, h2_ref):
    ids = ids_ref[...]
    bdim, tdim = ids.shape
    iota = jax.lax.broadcasted_iota(jnp.int32, (bdim, tdim, 128), 2)
    onehot = (ids[:, :, None] == iota).astype(jnp.float32)
    dn = (((2,), (0,)), ((), ()))
    h1_ref[...] = jax.lax.dot_general(onehot, t1_ref[...], dn,
                                      preferred_element_type=jnp.float32)
    h2_ref[...] = jax.lax.dot_general(onehot, t2_ref[...], dn,
                                      preferred_element_type=jnp.float32)


def _sc_gather_body(ids_hbm, table_hbm, out_hbm, idx_v,
                    rows0, rows1, rows2, rows3,
                    g0, g1, g2, g3, s0, s1, s2, s3):
    info = plsc.get_sparse_core_info()
    nw = info.num_cores * info.num_subcores
    wid = lax.axis_index("s") * info.num_cores + lax.axis_index("c")
    tokens_per_w = out_hbm.shape[0] // nw
    nch = tokens_per_w // _CH
    base = wid * tokens_per_w
    pltpu.sync_copy(ids_hbm.at[pl.ds(base, tokens_per_w)], idx_v)
    rows = (rows0, rows1, rows2, rows3)
    gsem = (g0, g1, g2, g3)
    ssem = (s0, s1, s2, s3)
    gcp = [None] * nch
    scp = [None] * nch
    for ch in range(nch):
        b = ch % 4
        if ch >= 4:
            scp[ch - 4].wait()
        gcp[ch] = pltpu.async_copy(
            table_hbm.at[idx_v.at[pl.ds(ch * _CH, _CH)]], rows[b], gsem[b])
        if ch >= 1:
            pb = (ch - 1) % 4
            gcp[ch - 1].wait()
            scp[ch - 1] = pltpu.async_copy(
                rows[pb], out_hbm.at[pl.ds(base + (ch - 1) * _CH, _CH)], ssem[pb])
    gcp[nch - 1].wait()
    scp[nch - 1] = pltpu.async_copy(
        rows[(nch - 1) % 4], out_hbm.at[pl.ds(base + (nch - 1) * _CH, _CH)],
        ssem[(nch - 1) % 4])
    for ch in range(max(nch - 4, 0), nch):
        scp[ch].wait()


def kernel(input_ids, emb, W1, b1, W2, b2):
    B, S = input_ids.shape
    V, H = emb.shape
    embp = jnp.zeros((128, H), dtype=emb.dtype).at[:V].set(emb)
    w1t = W1.T
    w2t = W2.T
    b1r = b1.reshape(1, H)
    b2r = b2.reshape(1, H)

    full = lambda: (0, 0)
    t1, t2 = pl.pallas_call(
        _tables_body,
        out_shape=[jax.ShapeDtypeStruct((128, H), jnp.float32)] * 2,
    )(embp, w1t, b1r, w2t, b2r)

    ids_flat = input_ids.reshape(B * S)
    mesh = plsc.VectorSubcoreMesh(core_axis_name="c", subcore_axis_name="s")
    sc_gather = functools.partial(
        pl.kernel,
        out_type=jax.ShapeDtypeStruct((B * S, H), jnp.float32),
        mesh=mesh,
        scratch_types=(
            [pltpu.VMEM((B * S // 32,), jnp.int32)]
            + [pltpu.VMEM((_CH, H), jnp.float32)] * 4
            + [pltpu.SemaphoreType.DMA] * 8
        ),
    )(_sc_gather_body)
    x = sc_gather(ids_flat, embp).reshape(B, S, H)

    nblk = S // _TB
    grid_spec = pl.GridSpec(
        grid=(nblk,),
        in_specs=[
            pl.BlockSpec((B, _TB), lambda i: (0, i)),
            pl.BlockSpec((128, H), lambda i: (0, 0)),
            pl.BlockSpec((128, H), lambda i: (0, 0)),
        ],
        out_specs=[
            pl.BlockSpec((B, _TB, H), lambda i: (0, i, 0)),
            pl.BlockSpec((B, _TB, H), lambda i: (0, i, 0)),
        ],
    )
    h1, h2 = pl.pallas_call(
        _hid_body,
        grid_spec=grid_spec,
        out_shape=[jax.ShapeDtypeStruct((B, S, H), jnp.float32)] * 2,
        compiler_params=pltpu.CompilerParams(
            dimension_semantics=("arbitrary",),
        ),
    )(input_ids, t1, t2)
    return (x, h1, h2)


# speedup vs baseline: 3.0143x; 3.0143x over previous
import jax
import jax.numpy as jnp
from jax.experimental import pallas as pl
from jax.experimental.pallas import tpu as pltpu

_TB = 2048


def _fused_body(ids_ref, emb_ref, w1t_ref, b1_ref, w2t_ref, b2_ref,
                x_ref, h1_ref, h2_ref):
    ids = ids_ref[...]
    bdim, tdim = ids.shape
    iota = jax.lax.broadcasted_iota(jnp.int32, (bdim, tdim, 128), 2)
    onehot = (ids[:, :, None] == iota).astype(jnp.float32)
    dn = (((2,), (0,)), ((), ()))
    x = jax.lax.dot_general(onehot, emb_ref[...], dn,
                            preferred_element_type=jnp.float32)
    x_ref[...] = x
    b1 = b1_ref[0][None, None, :]
    h1 = jax.lax.dot_general(x, w1t_ref[...], dn,
                             preferred_element_type=jnp.float32) + b1
    h1_ref[...] = h1
    b2 = b2_ref[0][None, None, :]
    h2 = jax.lax.dot_general(h1, w2t_ref[...], dn,
                             preferred_element_type=jnp.float32) + b2
    h2_ref[...] = h2


def kernel(input_ids, emb, W1, b1, W2, b2):
    B, S = input_ids.shape
    V, H = emb.shape
    embp = jnp.zeros((128, H), dtype=emb.dtype).at[:V].set(emb)
    w1t = W1.T
    w2t = W2.T
    b1r = b1.reshape(1, H)
    b2r = b2.reshape(1, H)

    nblk = S // _TB
    full = lambda i: (0, 0)
    grid_spec = pl.GridSpec(
        grid=(nblk,),
        in_specs=[
            pl.BlockSpec((B, _TB), lambda i: (0, i)),
            pl.BlockSpec((128, H), full),
            pl.BlockSpec((H, H), full),
            pl.BlockSpec((1, H), full),
            pl.BlockSpec((H, H), full),
            pl.BlockSpec((1, H), full),
        ],
        out_specs=[
            pl.BlockSpec((B, _TB, H), lambda i: (0, i, 0)),
            pl.BlockSpec((B, _TB, H), lambda i: (0, i, 0)),
            pl.BlockSpec((B, _TB, H), lambda i: (0, i, 0)),
        ],
    )
    out_shape = [jax.ShapeDtypeStruct((B, S, H), jnp.float32)] * 3
    x, h1, h2 = pl.pallas_call(
        _fused_body,
        grid_spec=grid_spec,
        out_shape=out_shape,
        compiler_params=pltpu.CompilerParams(
            dimension_semantics=("arbitrary",),
        ),
    )(input_ids, embp, w1t, b1r, w2t, b2r)
    return (x, h1, h2)
